# baseline (speedup 1.0000x reference)
import functools

import jax
import jax.numpy as jnp
from jax import lax
from jax.experimental import pallas as pl
from jax.experimental.pallas import tpu as pltpu
from jax.experimental.pallas import tpu_sc as plsc

K = 1024
D = 64
BETA = 0.25
BLOCK_T = 1024

_NC = 2
_NS = 16
_NW = _NC * _NS
_CH = 128


def _rowsum64(s):
    acc = s[:, 0:8]
    for k in range(1, 8):
        acc = acc + s[:, 8 * k:8 * (k + 1)]
    acc = acc[:, :4] + acc[:, 4:]
    acc = acc[:, :2] + acc[:, 2:]
    acc = acc[:, :1] + acc[:, 1:]
    return acc


def _tc_argmin_kernel(z_ref, z2_ref, e_ref, e2_ref, inds_ref, inds_row_ref,
                      loss_ref, et_ref):
    pid = pl.program_id(0)

    @pl.when(pid == 0)
    def _init():
        et_ref[...] = e_ref[...].T
        loss_ref[...] = jnp.zeros((1, 1), jnp.float32)

    z = z_ref[...]
    z2 = z2_ref[...]
    mm2 = jnp.dot(z * -2.0, et_ref[...], preferred_element_type=jnp.float32)
    GC = 128
    e2 = e2_ref[...]
    best = (z2 + e2[:, 0:GC]) + mm2[:, 0:GC]
    bidx = jnp.zeros((BLOCK_T, GC), jnp.int32)
    for g in range(1, K // GC):
        d_g = (z2 + e2[:, g * GC:(g + 1) * GC]) + mm2[:, g * GC:(g + 1) * GC]
        upd = d_g < best
        best = jnp.where(upd, d_g, best)
        bidx = jnp.where(upd, g, bidx)
    mins = jnp.min(best, axis=1, keepdims=True)
    lane = lax.broadcasted_iota(jnp.int32, (BLOCK_T, GC), 1)
    absidx = bidx * GC + lane
    inds = jnp.min(jnp.where(best == mins, absidx, K), axis=1).astype(jnp.int32)
    inds_ref[...] = inds.reshape(BLOCK_T // 128, 128)
    inds_row_ref[...] = inds[None, :]
    loss_ref[...] += jnp.sum(mins).reshape(1, 1)


def _sc_gather_body(table_hbm, idx_hbm, out_hbm, idx_v, rows_v, sem):
    bpw = idx_v.shape[0] * _CH
    nch = idx_v.shape[0]
    wid = lax.axis_index("s") * _NC + lax.axis_index("c")
    base = wid * bpw
    pltpu.sync_copy(idx_hbm.at[pl.ds(wid * nch, nch)], idx_v)
    copies = [
        pltpu.async_copy(table_hbm.at[idx_v.at[j]],
                         rows_v.at[pl.ds(j * _CH, _CH)], sem)
        for j in range(nch)
    ]
    for c in copies:
        c.wait()
    pltpu.sync_copy(rows_v, out_hbm.at[pl.ds(base, bpw)])


def _sc_gather(embedding, inds_lin, n):
    bpw = n // _NW
    nch = bpw // _CH
    mesh = plsc.VectorSubcoreMesh(core_axis_name="c", subcore_axis_name="s")
    run = functools.partial(
        pl.kernel,
        out_type=jax.ShapeDtypeStruct((n, D), jnp.float32),
        mesh=mesh,
        scratch_types=[
            pltpu.VMEM((nch, _CH), jnp.int32),
            pltpu.VMEM((bpw, D), jnp.float32),
            pltpu.SemaphoreType.DMA,
        ],
        compiler_params=pltpu.CompilerParams(use_tc_tiling_on_sc=False),
    )(_sc_gather_body)
    return run(embedding, inds_lin)


def kernel(latents, embedding, epc):
    b, t, d = latents.shape
    n = b * t
    flat = latents.reshape(n, d)
    num_blocks = n // BLOCK_T

    z2 = jnp.sum(flat * flat, axis=1, keepdims=True)
    e2 = jnp.sum(embedding * embedding, axis=1)[None, :]

    inds_lin, inds_row, loss_sum = pl.pallas_call(
        _tc_argmin_kernel,
        grid=(num_blocks,),
        in_specs=[
            pl.BlockSpec((BLOCK_T, D), lambda i: (i, 0)),
            pl.BlockSpec((BLOCK_T, 1), lambda i: (i, 0)),
            pl.BlockSpec((K, D), lambda i: (0, 0)),
            pl.BlockSpec((1, K), lambda i: (0, 0)),
        ],
    out_specs=[
            pl.BlockSpec((BLOCK_T // 128, 128), lambda i: (i, 0)),
            pl.BlockSpec((1, BLOCK_T), lambda i: (0, i)),
            pl.BlockSpec((1, 1), lambda i: (0, 0)),
        ],
        out_shape=[
            jax.ShapeDtypeStruct((n // 128, 128), jnp.int32),
            jax.ShapeDtypeStruct((1, n), jnp.int32),
            jax.ShapeDtypeStruct((1, 1), jnp.float32),
        ],
        scratch_shapes=[pltpu.VMEM((D, K), jnp.float32)],
    )(flat, z2, embedding, e2)

    q = _sc_gather(embedding, inds_lin, n)
    quantized_st = q.reshape(b, t, d)
    vq_loss = loss_sum[0, 0] * ((1.0 + BETA) / n / d)
    return (quantized_st, vq_loss, inds_row)


# speedup vs baseline: 1.2841x; 1.2841x over previous
import functools

import jax
import jax.numpy as jnp
from jax import lax
from jax.experimental import pallas as pl
from jax.experimental.pallas import tpu as pltpu
from jax.experimental.pallas import tpu_sc as plsc

K = 1024
D = 64
BETA = 0.25
BLOCK_T = 1024

_NC = 2
_NS = 16
_NW = _NC * _NS
_CH = 128


def _rowsum64(s):
    acc = s[:, 0:8]
    for k in range(1, 8):
        acc = acc + s[:, 8 * k:8 * (k + 1)]
    acc = acc[:, :4] + acc[:, 4:]
    acc = acc[:, :2] + acc[:, 2:]
    acc = acc[:, :1] + acc[:, 1:]
    return acc


def _tc_argmin_kernel(z_ref, z2_ref, e_ref, e2_ref, inds_ref, loss_ref, et_ref):
    pid = pl.program_id(0)

    @pl.when(pid == 0)
    def _init():
        et_ref[...] = e_ref[...].T
        loss_ref[...] = jnp.zeros((1, 1), jnp.float32)

    z = z_ref[...]
    z2 = z2_ref[...]
    mm2 = jnp.dot(z * -2.0, et_ref[...], preferred_element_type=jnp.float32)
    GC = 128
    e2 = e2_ref[...]
    best = (z2 + e2[:, 0:GC]) + mm2[:, 0:GC]
    bidx = jnp.zeros((BLOCK_T, GC), jnp.int32)
    for g in range(1, K // GC):
        d_g = (z2 + e2[:, g * GC:(g + 1) * GC]) + mm2[:, g * GC:(g + 1) * GC]
        upd = d_g < best
        best = jnp.where(upd, d_g, best)
        bidx = jnp.where(upd, g, bidx)
    mins = jnp.min(best, axis=1, keepdims=True)
    lane = lax.broadcasted_iota(jnp.int32, (BLOCK_T, GC), 1)
    absidx = bidx * GC + lane
    inds = jnp.min(jnp.where(best == mins, absidx, K), axis=1).astype(jnp.int32)
    inds_ref[...] = inds.reshape(BLOCK_T // 128, 128)
    loss_ref[...] += jnp.sum(mins).reshape(1, 1)


def _sc_gather_body(table_hbm, idx_hbm, out_hbm, idx_v, rows_v, sem):
    bpw = idx_v.shape[0] * _CH
    nch = idx_v.shape[0]
    wid = lax.axis_index("s") * _NC + lax.axis_index("c")
    base = wid * bpw
    pltpu.sync_copy(idx_hbm.at[pl.ds(wid * nch, nch)], idx_v)
    copies = [
        pltpu.async_copy(table_hbm.at[idx_v.at[j]],
                         rows_v.at[pl.ds(j * _CH, _CH)], sem)
        for j in range(nch)
    ]
    for c in copies:
        c.wait()
    pltpu.sync_copy(rows_v, out_hbm.at[pl.ds(base, bpw)])


def _sc_gather(embedding, inds_lin, n):
    bpw = n // _NW
    nch = bpw // _CH
    mesh = plsc.VectorSubcoreMesh(core_axis_name="c", subcore_axis_name="s")
    run = functools.partial(
        pl.kernel,
        out_type=jax.ShapeDtypeStruct((n, D), jnp.float32),
        mesh=mesh,
        scratch_types=[
            pltpu.VMEM((nch, _CH), jnp.int32),
            pltpu.VMEM((bpw, D), jnp.float32),
            pltpu.SemaphoreType.DMA,
        ],
        compiler_params=pltpu.CompilerParams(use_tc_tiling_on_sc=False),
    )(_sc_gather_body)
    return run(embedding, inds_lin)


def kernel(latents, embedding, epc):
    b, t, d = latents.shape
    n = b * t
    flat = latents.reshape(n, d)
    num_blocks = n // BLOCK_T

    z2 = jnp.sum(flat * flat, axis=1, keepdims=True)
    e2 = jnp.sum(embedding * embedding, axis=1)[None, :]

    inds_lin, loss_sum = pl.pallas_call(
        _tc_argmin_kernel,
        grid=(num_blocks,),
        in_specs=[
            pl.BlockSpec((BLOCK_T, D), lambda i: (i, 0)),
            pl.BlockSpec((BLOCK_T, 1), lambda i: (i, 0)),
            pl.BlockSpec((K, D), lambda i: (0, 0)),
            pl.BlockSpec((1, K), lambda i: (0, 0)),
        ],
    out_specs=[
            pl.BlockSpec((BLOCK_T // 128, 128), lambda i: (i, 0)),
            pl.BlockSpec((1, 1), lambda i: (0, 0)),
        ],
        out_shape=[
            jax.ShapeDtypeStruct((n // 128, 128), jnp.int32),
            jax.ShapeDtypeStruct((1, 1), jnp.float32),
        ],
        scratch_shapes=[pltpu.VMEM((D, K), jnp.float32)],
    )(flat, z2, embedding, e2)

    q = _sc_gather(embedding, inds_lin, n)
    quantized_st = q.reshape(b, t, d)
    vq_loss = loss_sum[0, 0] * ((1.0 + BETA) / n / d)
    return (quantized_st, vq_loss, inds_lin.reshape(1, n))


# speedup vs baseline: 1.3320x; 1.0373x over previous
import functools

import jax
import jax.numpy as jnp
from jax import lax
from jax.experimental import pallas as pl
from jax.experimental.pallas import tpu as pltpu
from jax.experimental.pallas import tpu_sc as plsc

K = 1024
D = 64
BETA = 0.25
BLOCK_T = 2048

_NC = 2
_NS = 16
_NW = _NC * _NS
_CH = 128


def _rowsum64(s):
    acc = s[:, 0:8]
    for k in range(1, 8):
        acc = acc + s[:, 8 * k:8 * (k + 1)]
    acc = acc[:, :4] + acc[:, 4:]
    acc = acc[:, :2] + acc[:, 2:]
    acc = acc[:, :1] + acc[:, 1:]
    return acc


def _tc_argmin_kernel(z_ref, z2_ref, e_ref, e2_ref, inds_ref, loss_ref, et_ref):
    pid = pl.program_id(0)

    @pl.when(pid == 0)
    def _init():
        et_ref[...] = e_ref[...].T
        loss_ref[...] = jnp.zeros((1, 1), jnp.float32)

    z = z_ref[...]
    z2 = z2_ref[...]
    mm2 = jnp.dot(z * -2.0, et_ref[...], preferred_element_type=jnp.float32)
    GC = 128
    e2 = e2_ref[...]
    best = (z2 + e2[:, 0:GC]) + mm2[:, 0:GC]
    bidx = jnp.zeros((BLOCK_T, GC), jnp.int32)
    for g in range(1, K // GC):
        d_g = (z2 + e2[:, g * GC:(g + 1) * GC]) + mm2[:, g * GC:(g + 1) * GC]
        upd = d_g < best
        best = jnp.where(upd, d_g, best)
        bidx = jnp.where(upd, g, bidx)
    mins = jnp.min(best, axis=1, keepdims=True)
    lane = lax.broadcasted_iota(jnp.int32, (BLOCK_T, GC), 1)
    absidx = bidx * GC + lane
    inds = jnp.min(jnp.where(best == mins, absidx, K), axis=1).astype(jnp.int32)
    inds_ref[...] = inds.reshape(BLOCK_T // 128, 128)
    loss_ref[...] += jnp.sum(mins).reshape(1, 1)


def _sc_gather_body(table_hbm, idx_hbm, out_hbm, idx_v, rows_v, sem):
    bpw = idx_v.shape[0] * _CH
    nch = idx_v.shape[0]
    wid = lax.axis_index("s") * _NC + lax.axis_index("c")
    base = wid * bpw
    pltpu.sync_copy(idx_hbm.at[pl.ds(wid * nch, nch)], idx_v)
    copies = [
        pltpu.async_copy(table_hbm.at[idx_v.at[j]],
                         rows_v.at[pl.ds(j * _CH, _CH)], sem)
        for j in range(nch)
    ]
    for c in copies:
        c.wait()
    pltpu.sync_copy(rows_v, out_hbm.at[pl.ds(base, bpw)])


def _sc_gather(embedding, inds_lin, n):
    bpw = n // _NW
    nch = bpw // _CH
    mesh = plsc.VectorSubcoreMesh(core_axis_name="c", subcore_axis_name="s")
    run = functools.partial(
        pl.kernel,
        out_type=jax.ShapeDtypeStruct((n, D), jnp.float32),
        mesh=mesh,
        scratch_types=[
            pltpu.VMEM((nch, _CH), jnp.int32),
            pltpu.VMEM((bpw, D), jnp.float32),
            pltpu.SemaphoreType.DMA,
        ],
        compiler_params=pltpu.CompilerParams(use_tc_tiling_on_sc=False),
    )(_sc_gather_body)
    return run(embedding, inds_lin)


def kernel(latents, embedding, epc):
    b, t, d = latents.shape
    n = b * t
    flat = latents.reshape(n, d)
    num_blocks = n // BLOCK_T

    z2 = jnp.sum(flat * flat, axis=1, keepdims=True)
    e2 = jnp.sum(embedding * embedding, axis=1)[None, :]

    inds_lin, loss_sum = pl.pallas_call(
        _tc_argmin_kernel,
        grid=(num_blocks,),
        in_specs=[
            pl.BlockSpec((BLOCK_T, D), lambda i: (i, 0)),
            pl.BlockSpec((BLOCK_T, 1), lambda i: (i, 0)),
            pl.BlockSpec((K, D), lambda i: (0, 0)),
            pl.BlockSpec((1, K), lambda i: (0, 0)),
        ],
    out_specs=[
            pl.BlockSpec((BLOCK_T // 128, 128), lambda i: (i, 0)),
            pl.BlockSpec((1, 1), lambda i: (0, 0)),
        ],
        out_shape=[
            jax.ShapeDtypeStruct((n // 128, 128), jnp.int32),
            jax.ShapeDtypeStruct((1, 1), jnp.float32),
        ],
        scratch_shapes=[pltpu.VMEM((D, K), jnp.float32)],
    )(flat, z2, embedding, e2)

    q = _sc_gather(embedding, inds_lin, n)
    quantized_st = q.reshape(b, t, d)
    vq_loss = loss_sum[0, 0] * ((1.0 + BETA) / n / d)
    return (quantized_st, vq_loss, inds_lin.reshape(1, n))
